# baseline (speedup 1.0000x reference)
import jax
import jax.numpy as jnp
from jax import lax
from jax.experimental import pallas as pl
from jax.experimental.pallas import tpu as pltpu
from jax.experimental.pallas import tpu_sc as plsc

_B, _F = 4096, 26
_D = 64
_NC, _NS = 2, 16
_NW = _NC * _NS
_TOTAL = _B * _F
_HALF = _TOTAL // 2
_PER_W = _TOTAL // _NW
_CHUNK = 128
_NCH = _PER_W // _CHUNK
_NBUF = 5


def _gather_body(ids_hbm, t2_hbm, out_hbm, idx_v, qidx_v, rows_v, outbuf_v,
                 *sems):
    gsem = sems[:_NBUF]
    osem = sems[_NBUF:]
    wid = lax.axis_index("s") * _NC + lax.axis_index("c")
    base = wid * _PER_W
    pltpu.sync_copy(ids_hbm.at[wid], idx_v)

    def prep(j, _):
        for v in range(8):
            qidx_v[j, pl.ds(v * 16, 16)] = (
                idx_v[j, pl.ds(v * 16, 16)] >> jnp.int32(1)
            )
        return 0
    lax.fori_loop(0, _NCH, prep, 0, unroll=False)

    for b in range(_NBUF):
        pltpu.async_copy(t2_hbm.at[qidx_v.at[b]], rows_v.at[b], gsem[b])

    tail = []
    for j in range(_NCH):
        b = j % _NBUF
        ob = j % 2
        pltpu.make_async_copy(
            t2_hbm.at[qidx_v.at[j]], rows_v.at[b], gsem[b]
        ).wait()

        def emit(m, _):
            par = (idx_v[j, pl.ds(m * 16, 16)] & jnp.int32(1)) * jnp.int32(_D)
            for t in range(16):
                off = pl.multiple_of(par[t], _D)
                for q in range(4):
                    outbuf_v[ob, m * 16 + t, pl.ds(16 * q, 16)] = rows_v[
                        b, m * 16 + t, pl.ds(off + 16 * q, 16)
                    ]
            return 0
        lax.fori_loop(0, 8, emit, 0, unroll=False)

        out_slice = out_hbm.at[pl.ds(base + j * _CHUNK, _CHUNK)]
        if j >= 2:
            pltpu.make_async_copy(
                out_hbm.at[pl.ds(0, _CHUNK)], outbuf_v.at[ob], osem[ob]
            ).wait()
        pltpu.async_copy(outbuf_v.at[ob], out_slice, osem[ob])
        nj = j + _NBUF
        if nj < _NCH:
            pltpu.async_copy(t2_hbm.at[qidx_v.at[nj]], rows_v.at[b], gsem[b])

    for ob in range(2):
        pltpu.make_async_copy(
            out_hbm.at[pl.ds(0, _CHUNK)], outbuf_v.at[ob], osem[ob]
        ).wait()


def _build():
    mesh = plsc.VectorSubcoreMesh(core_axis_name="c", subcore_axis_name="s")
    return pl.kernel(
        _gather_body,
        mesh=mesh,
        out_type=jax.ShapeDtypeStruct((_TOTAL, _D), jnp.float32),
        scratch_types=[
            pltpu.VMEM((_NCH, _CHUNK), jnp.int32),
            pltpu.VMEM((_NCH, _CHUNK), jnp.int32),
            pltpu.VMEM((_NBUF, _CHUNK, 2 * _D), jnp.float32),
            pltpu.VMEM((2, _CHUNK, _D), jnp.float32),
        ] + [pltpu.SemaphoreType.DMA] * (_NBUF + 2),
    )


_TCB = 1024


def _pack_body(tt_ref, o_ref):
    z = tt_ref[...].T.reshape(_TCB // 2, 2, _D)
    o_ref[:, 0:_D] = z[:, 0, :]
    o_ref[:, _D : 2 * _D] = z[:, 1, :]


def _pack():
    return pl.pallas_call(
        _pack_body,
        grid=((1000000 + _TCB - 1) // _TCB,),
        in_specs=[pl.BlockSpec((_D, _TCB), lambda i: (0, i))],
        out_specs=pl.BlockSpec((_TCB // 2, 128), lambda i: (i, 0)),
        out_shape=jax.ShapeDtypeStruct((500000, 128), jnp.float32),
    )


@jax.jit
def kernel(ids, table):
    ids3 = ids.reshape(_NW, _NCH, _CHUNK)
    t2 = _pack()(table.T)
    out = _build()(ids3, t2)
    return out.reshape(_B, _F, _D)


# speedup vs baseline: 1.5231x; 1.5231x over previous
import jax
import jax.numpy as jnp
from jax import lax
from jax.experimental import pallas as pl
from jax.experimental.pallas import tpu as pltpu
from jax.experimental.pallas import tpu_sc as plsc

_B, _F = 4096, 26
_D = 64
_NC, _NS = 2, 16
_NW = _NC * _NS
_TOTAL = _B * _F
_HALF = _TOTAL // 2
_PER_W = _TOTAL // _NW
_CHUNK = 128
_NCH = _PER_W // _CHUNK
_NBUF = 5


def _gather_body(ids_hbm, t2_hbm, out_hbm, idx_v, qidx_v, rows_v, outbuf_v,
                 *sems):
    gsem = sems[:_NBUF]
    osem = sems[_NBUF:]
    wid = lax.axis_index("s") * _NC + lax.axis_index("c")
    base = wid * _PER_W
    pltpu.sync_copy(ids_hbm.at[wid], idx_v)

    def prep(j, _):
        for v in range(8):
            qidx_v[j, pl.ds(v * 16, 16)] = (
                idx_v[j, pl.ds(v * 16, 16)] >> jnp.int32(1)
            )
        return 0
    lax.fori_loop(0, _NCH, prep, 0, unroll=False)

    for b in range(_NBUF):
        pltpu.async_copy(t2_hbm.at[qidx_v.at[b]], rows_v.at[b], gsem[b])

    tail = []
    for j in range(_NCH):
        b = j % _NBUF
        ob = j % 2
        pltpu.make_async_copy(
            t2_hbm.at[qidx_v.at[j]], rows_v.at[b], gsem[b]
        ).wait()

        def emit(m, _):
            par = (idx_v[j, pl.ds(m * 16, 16)] & jnp.int32(1)) * jnp.int32(_D)
            for t in range(16):
                off = pl.multiple_of(par[t], _D)
                for q in range(4):
                    outbuf_v[ob, m * 16 + t, pl.ds(16 * q, 16)] = rows_v[
                        b, m * 16 + t, pl.ds(off + 16 * q, 16)
                    ]
            return 0
        lax.fori_loop(0, 8, emit, 0, unroll=False)

        out_slice = out_hbm.at[pl.ds(base + j * _CHUNK, _CHUNK)]
        if j >= 2:
            pltpu.make_async_copy(
                out_hbm.at[pl.ds(0, _CHUNK)], outbuf_v.at[ob], osem[ob]
            ).wait()
        pltpu.async_copy(outbuf_v.at[ob], out_slice, osem[ob])
        nj = j + _NBUF
        if nj < _NCH:
            pltpu.async_copy(t2_hbm.at[qidx_v.at[nj]], rows_v.at[b], gsem[b])

    for ob in range(2):
        pltpu.make_async_copy(
            out_hbm.at[pl.ds(0, _CHUNK)], outbuf_v.at[ob], osem[ob]
        ).wait()


def _build():
    mesh = plsc.VectorSubcoreMesh(core_axis_name="c", subcore_axis_name="s")
    return pl.kernel(
        _gather_body,
        mesh=mesh,
        out_type=jax.ShapeDtypeStruct((_TOTAL, _D), jnp.float32),
        scratch_types=[
            pltpu.VMEM((_NCH, _CHUNK), jnp.int32),
            pltpu.VMEM((_NCH, _CHUNK), jnp.int32),
            pltpu.VMEM((_NBUF, _CHUNK, 2 * _D), jnp.float32),
            pltpu.VMEM((2, _CHUNK, _D), jnp.float32),
        ] + [pltpu.SemaphoreType.DMA] * (_NBUF + 2),
    )


_TCB = 4096


def _pack_body(tt_ref, o_ref):
    xt = jax.lax.dot_general(
        tt_ref[...], jnp.eye(_D, dtype=jnp.float32),
        (((0,), (0,)), ((), ())),
        preferred_element_type=jnp.float32,
    )
    z = xt.reshape(_TCB // 2, 2, _D)
    o_ref[:, 0:_D] = z[:, 0, :]
    o_ref[:, _D : 2 * _D] = z[:, 1, :]


def _pack():
    return pl.pallas_call(
        _pack_body,
        grid=((1000000 + _TCB - 1) // _TCB,),
        in_specs=[pl.BlockSpec((_D, _TCB), lambda i: (0, i))],
        out_specs=pl.BlockSpec((_TCB // 2, 128), lambda i: (i, 0)),
        out_shape=jax.ShapeDtypeStruct((500000, 128), jnp.float32),
    )


@jax.jit
def kernel(ids, table):
    ids3 = ids.reshape(_NW, _NCH, _CHUNK)
    t2 = _pack()(table.T)
    out = _build()(ids3, t2)
    return out.reshape(_B, _F, _D)


# speedup vs baseline: 1.9372x; 1.2719x over previous
import jax
import jax.numpy as jnp
from jax import lax
from jax.experimental import pallas as pl
from jax.experimental.pallas import tpu as pltpu
from jax.experimental.pallas import tpu_sc as plsc

_B, _F = 4096, 26
_D = 64
_NC, _NS = 2, 16
_NW = _NC * _NS
_TOTAL = _B * _F
_HALF = _TOTAL // 2
_PER_W = _TOTAL // _NW
_CHUNK = 128
_NCH = _PER_W // _CHUNK
_NBUF = 5


def _gather_body(ids_hbm, t2_hbm, out_hbm, idx_v, qidx_v, rows_v, outbuf_v,
                 *sems):
    gsem = sems[:_NBUF]
    osem = sems[_NBUF:]
    wid = lax.axis_index("s") * _NC + lax.axis_index("c")
    base = wid * _PER_W
    pltpu.sync_copy(ids_hbm.at[wid], idx_v)

    def prep(j, _):
        for v in range(8):
            r = idx_v[j, pl.ds(v * 16, 16)]
            qidx_v[j, pl.ds(v * 16, 16)] = (
                ((r >> jnp.int32(8)) << jnp.int32(7)) | (r & jnp.int32(127))
            )
        return 0
    lax.fori_loop(0, _NCH, prep, 0, unroll=False)

    for b in range(_NBUF):
        pltpu.async_copy(t2_hbm.at[qidx_v.at[b]], rows_v.at[b], gsem[b])

    tail = []
    for j in range(_NCH):
        b = j % _NBUF
        ob = j % 2
        pltpu.make_async_copy(
            t2_hbm.at[qidx_v.at[j]], rows_v.at[b], gsem[b]
        ).wait()

        def emit(m, _):
            par = (
                (idx_v[j, pl.ds(m * 16, 16)] >> jnp.int32(7)) & jnp.int32(1)
            ) * jnp.int32(_D)
            for t in range(16):
                off = pl.multiple_of(par[t], _D)
                for q in range(4):
                    outbuf_v[ob, m * 16 + t, pl.ds(16 * q, 16)] = rows_v[
                        b, m * 16 + t, pl.ds(off + 16 * q, 16)
                    ]
            return 0
        lax.fori_loop(0, 8, emit, 0, unroll=False)

        out_slice = out_hbm.at[pl.ds(base + j * _CHUNK, _CHUNK)]
        if j >= 2:
            pltpu.make_async_copy(
                out_hbm.at[pl.ds(0, _CHUNK)], outbuf_v.at[ob], osem[ob]
            ).wait()
        pltpu.async_copy(outbuf_v.at[ob], out_slice, osem[ob])
        nj = j + _NBUF
        if nj < _NCH:
            pltpu.async_copy(t2_hbm.at[qidx_v.at[nj]], rows_v.at[b], gsem[b])

    for ob in range(2):
        pltpu.make_async_copy(
            out_hbm.at[pl.ds(0, _CHUNK)], outbuf_v.at[ob], osem[ob]
        ).wait()


def _build():
    mesh = plsc.VectorSubcoreMesh(core_axis_name="c", subcore_axis_name="s")
    return pl.kernel(
        _gather_body,
        mesh=mesh,
        out_type=jax.ShapeDtypeStruct((_TOTAL, _D), jnp.float32),
        scratch_types=[
            pltpu.VMEM((_NCH, _CHUNK), jnp.int32),
            pltpu.VMEM((_NCH, _CHUNK), jnp.int32),
            pltpu.VMEM((_NBUF, _CHUNK, 2 * _D), jnp.float32),
            pltpu.VMEM((2, _CHUNK, _D), jnp.float32),
        ] + [pltpu.SemaphoreType.DMA] * (_NBUF + 2),
    )


_TCB = 4096


def _pack_body(tt_ref, o_ref):
    xt = tt_ref[...].T
    for s in range(_TCB // 128):
        half = s % 2
        o_ref[
            pl.ds((s // 2) * 128, 128), pl.ds(half * _D, _D)
        ] = xt[s * 128 : (s + 1) * 128, :]


def _pack():
    return pl.pallas_call(
        _pack_body,
        grid=((1000000 + _TCB - 1) // _TCB,),
        in_specs=[pl.BlockSpec((_D, _TCB), lambda i: (0, i))],
        out_specs=pl.BlockSpec((_TCB // 2, 128), lambda i: (i, 0)),
        out_shape=jax.ShapeDtypeStruct((500032, 128), jnp.float32),
    )


@jax.jit
def kernel(ids, table):
    ids3 = ids.reshape(_NW, _NCH, _CHUNK)
    t2 = _pack()(table.T)
    out = _build()(ids3, t2)
    return out.reshape(_B, _F, _D)


# speedup vs baseline: 2.2595x; 1.1664x over previous
import jax
import jax.numpy as jnp
from jax import lax
from jax.experimental import pallas as pl
from jax.experimental.pallas import tpu as pltpu
from jax.experimental.pallas import tpu_sc as plsc

_B, _F = 4096, 26
_D = 64
_NC, _NS = 2, 16
_NW = _NC * _NS
_TOTAL = _B * _F
_HALF = _TOTAL // 2
_PER_W = _TOTAL // _NW
_CHUNK = 128
_NCH = _PER_W // _CHUNK
_NBUF = 5


def _gather_body(ids_hbm, t2_hbm, out_hbm, idx_v, qidx_v, rows_v, outbuf_v,
                 *sems):
    gsem = sems[:_NBUF]
    osem = sems[_NBUF:]
    wid = lax.axis_index("s") * _NC + lax.axis_index("c")
    base = wid * _PER_W
    pltpu.sync_copy(ids_hbm.at[wid], idx_v)

    def prep(j, _):
        for v in range(8):
            r = idx_v[j, pl.ds(v * 16, 16)]
            qidx_v[j, pl.ds(v * 16, 16)] = (
                ((r >> jnp.int32(8)) << jnp.int32(7)) | (r & jnp.int32(127))
            )
        return 0
    lax.fori_loop(0, _NCH, prep, 0, unroll=False)

    for b in range(_NBUF):
        pltpu.async_copy(t2_hbm.at[qidx_v.at[b]], rows_v.at[b], gsem[b])

    tail = []
    for j in range(_NCH):
        b = j % _NBUF
        ob = j % 2
        pltpu.make_async_copy(
            t2_hbm.at[qidx_v.at[j]], rows_v.at[b], gsem[b]
        ).wait()

        def emit(m, _):
            par = (
                (idx_v[j, pl.ds(m * 16, 16)] >> jnp.int32(7)) & jnp.int32(1)
            ) * jnp.int32(_D)
            for t in range(16):
                off = pl.multiple_of(par[t], _D)
                for q in range(4):
                    outbuf_v[ob, m * 16 + t, pl.ds(16 * q, 16)] = rows_v[
                        b, m * 16 + t, pl.ds(off + 16 * q, 16)
                    ]
            return 0
        lax.fori_loop(0, 8, emit, 0, unroll=False)

        out_slice = out_hbm.at[pl.ds(base + j * _CHUNK, _CHUNK)]
        if j >= 2:
            pltpu.make_async_copy(
                out_hbm.at[pl.ds(0, _CHUNK)], outbuf_v.at[ob], osem[ob]
            ).wait()
        pltpu.async_copy(outbuf_v.at[ob], out_slice, osem[ob])
        nj = j + _NBUF
        if nj < _NCH:
            pltpu.async_copy(t2_hbm.at[qidx_v.at[nj]], rows_v.at[b], gsem[b])

    for ob in range(2):
        pltpu.make_async_copy(
            out_hbm.at[pl.ds(0, _CHUNK)], outbuf_v.at[ob], osem[ob]
        ).wait()


def _build():
    mesh = plsc.VectorSubcoreMesh(core_axis_name="c", subcore_axis_name="s")
    return pl.kernel(
        _gather_body,
        mesh=mesh,
        out_type=jax.ShapeDtypeStruct((_TOTAL, _D), jnp.float32),
        scratch_types=[
            pltpu.VMEM((_NCH, _CHUNK), jnp.int32),
            pltpu.VMEM((_NCH, _CHUNK), jnp.int32),
            pltpu.VMEM((_NBUF, _CHUNK, 2 * _D), jnp.float32),
            pltpu.VMEM((2, _CHUNK, _D), jnp.float32),
        ] + [pltpu.SemaphoreType.DMA] * (_NBUF + 2),
    )


_TCB = 8192


def _pack_body(tt_ref, o_ref):
    xt = tt_ref[...].T
    for s in range(_TCB // 128):
        half = s % 2
        o_ref[
            pl.ds((s // 2) * 128, 128), pl.ds(half * _D, _D)
        ] = xt[s * 128 : (s + 1) * 128, :]


def _pack():
    return pl.pallas_call(
        _pack_body,
        grid=((1000000 + _TCB - 1) // _TCB,),
        in_specs=[pl.BlockSpec((_D, _TCB), lambda i: (0, i))],
        out_specs=pl.BlockSpec((_TCB // 2, 128), lambda i: (i, 0)),
        out_shape=jax.ShapeDtypeStruct((500032, 128), jnp.float32),
    )


@jax.jit
def kernel(ids, table):
    ids3 = ids.reshape(_NW, _NCH, _CHUNK)
    t2 = _pack()(table.T)
    out = _build()(ids3, t2)
    return out.reshape(_B, _F, _D)


# speedup vs baseline: 2.4463x; 1.0827x over previous
import jax
import jax.numpy as jnp
from jax import lax
from jax.experimental import pallas as pl
from jax.experimental.pallas import tpu as pltpu
from jax.experimental.pallas import tpu_sc as plsc

_B, _F = 4096, 26
_D = 64
_NC, _NS = 2, 16
_NW = _NC * _NS
_TOTAL = _B * _F
_HALF = _TOTAL // 2
_PER_W = _TOTAL // _NW
_CHUNK = 128
_NCH = _PER_W // _CHUNK
_NBUF = 5


def _gather_body(ids_hbm, t2_hbm, out_hbm, idx_v, qidx_v, rows_v, outbuf_v,
                 *sems):
    gsem = sems[:_NBUF]
    osem = sems[_NBUF:]
    wid = lax.axis_index("s") * _NC + lax.axis_index("c")
    base = wid * _PER_W
    pltpu.sync_copy(ids_hbm.at[wid], idx_v)

    def prep(j, _):
        for v in range(8):
            r = idx_v[j, pl.ds(v * 16, 16)]
            qidx_v[j, pl.ds(v * 16, 16)] = (
                ((r >> jnp.int32(8)) << jnp.int32(7)) | (r & jnp.int32(127))
            )
        return 0
    lax.fori_loop(0, _NCH, prep, 0, unroll=False)

    for b in range(_NBUF):
        pltpu.async_copy(t2_hbm.at[qidx_v.at[b]], rows_v.at[b], gsem[b])

    tail = []
    for j in range(_NCH):
        b = j % _NBUF
        ob = j % 2
        pltpu.make_async_copy(
            t2_hbm.at[qidx_v.at[j]], rows_v.at[b], gsem[b]
        ).wait()

        def emit(m, _):
            par = (
                (idx_v[j, pl.ds(m * 16, 16)] >> jnp.int32(7)) & jnp.int32(1)
            ) * jnp.int32(_D)
            for t in range(16):
                off = pl.multiple_of(par[t], _D)
                for q in range(4):
                    outbuf_v[ob, m * 16 + t, pl.ds(16 * q, 16)] = rows_v[
                        b, m * 16 + t, pl.ds(off + 16 * q, 16)
                    ]
            return 0
        lax.fori_loop(0, 8, emit, 0, unroll=False)

        out_slice = out_hbm.at[pl.ds(base + j * _CHUNK, _CHUNK)]
        if j >= 2:
            pltpu.make_async_copy(
                out_hbm.at[pl.ds(0, _CHUNK)], outbuf_v.at[ob], osem[ob]
            ).wait()
        pltpu.async_copy(outbuf_v.at[ob], out_slice, osem[ob])
        nj = j + _NBUF
        if nj < _NCH:
            pltpu.async_copy(t2_hbm.at[qidx_v.at[nj]], rows_v.at[b], gsem[b])

    for ob in range(2):
        pltpu.make_async_copy(
            out_hbm.at[pl.ds(0, _CHUNK)], outbuf_v.at[ob], osem[ob]
        ).wait()


def _build():
    mesh = plsc.VectorSubcoreMesh(core_axis_name="c", subcore_axis_name="s")
    return pl.kernel(
        _gather_body,
        mesh=mesh,
        out_type=jax.ShapeDtypeStruct((_TOTAL, _D), jnp.float32),
        scratch_types=[
            pltpu.VMEM((_NCH, _CHUNK), jnp.int32),
            pltpu.VMEM((_NCH, _CHUNK), jnp.int32),
            pltpu.VMEM((_NBUF, _CHUNK, 2 * _D), jnp.float32),
            pltpu.VMEM((2, _CHUNK, _D), jnp.float32),
        ] + [pltpu.SemaphoreType.DMA] * (_NBUF + 2),
    )


_TCB = 16384


def _pack_body(tt_ref, o_ref):
    xt = tt_ref[...].T
    for s in range(_TCB // 128):
        half = s % 2
        o_ref[
            pl.ds((s // 2) * 128, 128), pl.ds(half * _D, _D)
        ] = xt[s * 128 : (s + 1) * 128, :]


def _pack():
    return pl.pallas_call(
        _pack_body,
        grid=((1000000 + _TCB - 1) // _TCB,),
        in_specs=[pl.BlockSpec((_D, _TCB), lambda i: (0, i))],
        out_specs=pl.BlockSpec((_TCB // 2, 128), lambda i: (i, 0)),
        out_shape=jax.ShapeDtypeStruct((500032, 128), jnp.float32),
    )


@jax.jit
def kernel(ids, table):
    ids3 = ids.reshape(_NW, _NCH, _CHUNK)
    t2 = _pack()(table.T)
    out = _build()(ids3, t2)
    return out.reshape(_B, _F, _D)
